# baseline (speedup 1.0000x reference)
import functools
import jax
import jax.numpy as jnp
from jax import lax
from jax.experimental import pallas as pl
from jax.experimental.pallas import tpu as pltpu
from jax.experimental.pallas import tpu_sc as plsc

NUM_LEVELS_ = 8
K_ = 8192
D_ = 256
B_ = 4096
BT_ = 256
NBT_ = B_ // BT_
BR_ = 1024
NBR_ = B_ // BR_

_HI = jax.lax.Precision.HIGHEST
_BIG = 3e38


def _nt(a, b):
    return jax.lax.dot_general(a, b, (((1,), (1,)), ((), ())),
                               preferred_element_type=jnp.float32)


def _split3(x):
    x0 = x.astype(jnp.bfloat16)
    rem = x - x0.astype(jnp.float32)
    x1 = rem.astype(jnp.bfloat16)
    x2 = (rem - x1.astype(jnp.float32)).astype(jnp.bfloat16)
    return x0, x1, x2


def _scores_body(r_ref, cb_ref, ids1_ref, ids2_ref,
                 cnorm_scratch, c0_s, c1_s, c2_s):
    b = pl.program_id(0)

    @pl.when(b == 0)
    def _prep():
        cb = cb_ref[...]
        ones = jnp.ones((1, D_), jnp.float32)
        cnorm_scratch[...] = jax.lax.dot_general(
            ones, cb * cb, (((1,), (1,)), ((), ())),
            precision=_HI, preferred_element_type=jnp.float32)
        p0, p1, p2 = _split3(cb)
        c0_s[...] = p0
        c1_s[...] = p1
        c2_s[...] = p2

    r = r_ref[...]
    r0, r1, r2 = _split3(r)
    c0 = c0_s[...]
    c1 = c1_s[...]
    c2 = c2_s[...]
    rc = (_nt(r0, c0) + _nt(r0, c1) + _nt(r1, c0)
          + _nt(r0, c2) + _nt(r1, c1) + _nt(r2, c0))
    s = cnorm_scratch[...] - 2.0 * rc

    m1 = jnp.min(s, axis=1, keepdims=True)
    kiota = jax.lax.broadcasted_iota(jnp.int32, (BT_, K_), 1)
    idx1 = jnp.min(jnp.where(s == m1, kiota, K_), axis=1)
    s2 = jnp.where(kiota == idx1[:, None], _BIG, s)
    idx2 = jnp.argmin(s2, axis=1).astype(jnp.int32)
    ids1_ref[...] = idx1.reshape(1, 1, BT_)
    ids2_ref[...] = idx2.reshape(1, 1, BT_)


def _tc_scores_top2(r, cb_l):
    ids1, ids2 = pl.pallas_call(
        _scores_body,
        grid=(NBT_,),
        in_specs=[
            pl.BlockSpec((BT_, D_), lambda b: (b, 0)),
            pl.BlockSpec((K_, D_), lambda b: (0, 0)),
        ],
        out_specs=[
            pl.BlockSpec((1, 1, BT_), lambda b: (0, 0, b)),
            pl.BlockSpec((1, 1, BT_), lambda b: (0, 0, b)),
        ],
        out_shape=[
            jax.ShapeDtypeStruct((1, 1, B_), jnp.int32),
            jax.ShapeDtypeStruct((1, 1, B_), jnp.int32),
        ],
        scratch_shapes=[
            pltpu.VMEM((1, K_), jnp.float32),
            pltpu.VMEM((K_, D_), jnp.bfloat16),
            pltpu.VMEM((K_, D_), jnp.bfloat16),
            pltpu.VMEM((K_, D_), jnp.bfloat16),
        ],
    )(r, cb_l)
    return ids1.reshape(B_), ids2.reshape(B_)


def _make_sc_gather2():
    info = plsc.get_sparse_core_info()
    nw = info.num_cores * info.num_subcores
    bw = B_ // nw
    mesh = plsc.VectorSubcoreMesh(core_axis_name="c", subcore_axis_name="s")

    @functools.partial(
        pl.kernel, mesh=mesh,
        out_type=[
            jax.ShapeDtypeStruct((B_, D_), jnp.float32),
            jax.ShapeDtypeStruct((B_, D_), jnp.float32),
        ],
        scratch_types=[
            pltpu.VMEM((bw,), jnp.int32),
            pltpu.VMEM((bw,), jnp.int32),
            pltpu.VMEM((bw, D_), jnp.float32),
            pltpu.VMEM((bw, D_), jnp.float32),
            pltpu.SemaphoreType.DMA,
            pltpu.SemaphoreType.DMA,
        ],
    )
    def sc_gather2(cb_hbm, idx1_hbm, idx2_hbm, g1_hbm, g2_hbm,
                   i1_v, i2_v, rows1_v, rows2_v, sem1, sem2):
        wid = lax.axis_index("s") * info.num_cores + lax.axis_index("c")
        base = wid * bw
        pltpu.sync_copy(idx1_hbm.at[pl.ds(base, bw)], i1_v)
        pltpu.sync_copy(idx2_hbm.at[pl.ds(base, bw)], i2_v)
        cp1 = pltpu.async_copy(cb_hbm.at[i1_v], rows1_v, sem1)
        cp2 = pltpu.async_copy(cb_hbm.at[i2_v], rows2_v, sem2)
        cp1.wait()
        pltpu.sync_copy(rows1_v, g1_hbm.at[pl.ds(base, bw)])
        cp2.wait()
        pltpu.sync_copy(rows2_v, g2_hbm.at[pl.ds(base, bw)])

    return sc_gather2


def _rescore_body(r_ref, g1_ref, g2_ref, i1_ref, i2_ref, rout_ref, ids_ref):
    rp = r_ref[...]
    g1 = g1_ref[...]
    g2 = g2_ref[...]
    idx1 = i1_ref[0, 0, :]
    idx2 = i2_ref[0, 0, :]
    diff1 = rp - g1
    d1 = jnp.sqrt(jnp.maximum(jnp.sum(diff1 * diff1, axis=1), 0.0))
    diff2 = rp - g2
    d2 = jnp.sqrt(jnp.maximum(jnp.sum(diff2 * diff2, axis=1), 0.0))
    take2 = jnp.logical_or(d2 < d1,
                           jnp.logical_and(d2 == d1, idx2 < idx1))
    winner = jnp.where(take2, idx2, idx1)
    chosen = jnp.where(take2[:, None], g2, g1)
    rout_ref[...] = rp - chosen
    ids_ref[...] = winner.reshape(1, 1, BR_)


def _tc_rescore(r, g1, g2, ids1, ids2):
    r_new, win = pl.pallas_call(
        _rescore_body,
        grid=(NBR_,),
        in_specs=[
            pl.BlockSpec((BR_, D_), lambda b: (b, 0)),
            pl.BlockSpec((BR_, D_), lambda b: (b, 0)),
            pl.BlockSpec((BR_, D_), lambda b: (b, 0)),
            pl.BlockSpec((1, 1, BR_), lambda b: (0, 0, b)),
            pl.BlockSpec((1, 1, BR_), lambda b: (0, 0, b)),
        ],
        out_specs=[
            pl.BlockSpec((BR_, D_), lambda b: (b, 0)),
            pl.BlockSpec((1, 1, BR_), lambda b: (0, 0, b)),
        ],
        out_shape=[
            jax.ShapeDtypeStruct((B_, D_), jnp.float32),
            jax.ShapeDtypeStruct((1, 1, B_), jnp.int32),
        ],
    )(r, g1, g2, ids1.reshape(1, 1, B_), ids2.reshape(1, 1, B_))
    return r_new, win.reshape(B_)


def kernel(preference_vector, codebooks):
    sc_gather2 = _make_sc_gather2()
    r = preference_vector
    ids = []
    for l in range(NUM_LEVELS_):
        cb_l = codebooks[l]
        idx1, idx2 = _tc_scores_top2(r, cb_l)
        g1, g2 = sc_gather2(cb_l, idx1, idx2)
        r, win = _tc_rescore(r, g1, g2, idx1, idx2)
        ids.append(win)
    return jnp.stack(ids, axis=1), r
